# Initial kernel scaffold; baseline (speedup 1.0000x reference)
import jax
import jax.numpy as jnp
from jax.experimental import pallas as pl


def kernel(view_a_pos, view_a_neg, view_b_pos, view_b_neg, x, W, b):
    raise NotImplementedError("write your pallas kernel here")


# speedup vs baseline: 25.4581x; 25.4581x over previous
import functools

import jax
import jax.numpy as jnp
from jax import lax
from jax.experimental import pallas as pl
from jax.experimental.pallas import tpu as pltpu
from jax.experimental.pallas import tpu_sc as plsc

N = 10000
D = 128
E = 160000
NC = 2
NS = 16
NPAD = 10240
RPT = NPAD // NS
B = 80
NBLK = E // NS // B
RB = 1000

_mesh = plsc.VectorSubcoreMesh(
    core_axis_name="c", subcore_axis_name="s", num_cores=NC, num_subcores=NS
)


def _deg_sc(dst_t):
    @functools.partial(
        pl.kernel,
        out_type=jax.ShapeDtypeStruct((NC, NPAD), jnp.float32),
        mesh=_mesh,
        scratch_types=[
            pltpu.VMEM_SHARED((NPAD,), jnp.float32),
            pltpu.VMEM((NBLK, B), jnp.int32),
            pltpu.VMEM((B,), jnp.float32),
            pltpu.VMEM((RPT,), jnp.float32),
        ],
    )
    def run(dst_ref, deg_ref, deg_sh, idx_v, ones_v, stage_v):
        c = lax.axis_index("c")
        s = lax.axis_index("s")
        for i in range(B // 16):
            ones_v[pl.ds(i * 16, 16)] = jnp.ones((16,), jnp.float32)

        def zfill(i, carry):
            stage_v[pl.ds(i * 16, 16)] = jnp.zeros((16,), jnp.float32)
            return carry

        lax.fori_loop(0, RPT // 16, zfill, 0)
        pltpu.sync_copy(stage_v, deg_sh.at[pl.ds(s * RPT, RPT)])
        pltpu.sync_copy(dst_ref.at[c, s], idx_v)
        plsc.subcore_barrier()

        def body(j, carry):
            pltpu.sync_copy(ones_v, deg_sh.at[idx_v.at[j]], add=True)
            return carry

        lax.fori_loop(0, NBLK, body, 0)
        plsc.subcore_barrier()
        pltpu.sync_copy(deg_sh.at[pl.ds(s * RPT, RPT)], stage_v)
        pltpu.sync_copy(stage_v, deg_ref.at[c, pl.ds(s * RPT, RPT)])

    return run(dst_t)


def _agg_sc(src_t, dst_t, h2):
    @functools.partial(
        pl.kernel,
        out_type=jax.ShapeDtypeStruct((NC, NPAD, D), jnp.float32),
        mesh=_mesh,
        scratch_types=[
            pltpu.VMEM_SHARED((NPAD, D), jnp.float32),
            pltpu.VMEM((NBLK, B), jnp.int32),
            pltpu.VMEM((NBLK, B), jnp.int32),
            pltpu.VMEM((B, D), jnp.float32),
            pltpu.SemaphoreType.DMA,
        ],
    )
    def run(src_ref, dst_ref, h2_ref, agg_ref, acc_sh, sidx_v, didx_v, rows_v, sem):
        c = lax.axis_index("c")
        s = lax.axis_index("s")

        def zfill(r, carry):
            for k in range(D // 16):
                rows_v[r, pl.ds(k * 16, 16)] = jnp.zeros((16,), jnp.float32)
            return carry

        lax.fori_loop(0, B, zfill, 0)
        for t in range(RPT // B):
            pltpu.sync_copy(rows_v, acc_sh.at[pl.ds(s * RPT + t * B, B)])
        pltpu.sync_copy(src_ref.at[c, s], sidx_v)
        pltpu.sync_copy(dst_ref.at[c, s], didx_v)
        plsc.subcore_barrier()

        def body(j, carry):
            pltpu.async_copy(h2_ref.at[sidx_v.at[j]], rows_v, sem).wait()
            pltpu.sync_copy(rows_v, acc_sh.at[didx_v.at[j]], add=True)
            return carry

        lax.fori_loop(0, NBLK, body, 0)
        plsc.subcore_barrier()
        for t in range(RPT // B):
            pltpu.sync_copy(acc_sh.at[pl.ds(s * RPT + t * B, B)], rows_v)
            pltpu.sync_copy(rows_v, agg_ref.at[c, pl.ds(s * RPT + t * B, B)])

    return run(src_t, dst_t, h2)


def _h2_tc(x, W, deg):
    def body(x_ref, w_ref, deg_ref, h2_ref):
        h = jnp.dot(x_ref[...], w_ref[...], preferred_element_type=jnp.float32)
        dinv = lax.rsqrt(deg_ref[...] + 1.0)
        h2_ref[...] = h[None, :, :] * dinv

    return pl.pallas_call(
        body,
        grid=(N // RB,),
        in_specs=[
            pl.BlockSpec((RB, D), lambda i: (i, 0)),
            pl.BlockSpec((D, D), lambda i: (0, 0)),
            pl.BlockSpec((NC, RB, 1), lambda i: (0, i, 0)),
        ],
        out_specs=pl.BlockSpec((NC, RB, D), lambda i: (0, i, 0)),
        out_shape=jax.ShapeDtypeStruct((NC, N, D), jnp.float32),
    )(x, W, deg)


def _final_tc(agg, h2, deg, bias):
    def body(agg_ref, h2_ref, deg_ref, b_ref, oa_ref, ob_ref):
        dinv = lax.rsqrt(deg_ref[...] + 1.0)
        r = dinv * (agg_ref[...] + h2_ref[...]) + b_ref[...][None]
        r = jnp.maximum(r, 0.0)
        oa_ref[...] = r[0]
        ob_ref[...] = r[1]

    return pl.pallas_call(
        body,
        grid=(N // RB,),
        in_specs=[
            pl.BlockSpec((NC, RB, D), lambda i: (0, i, 0)),
            pl.BlockSpec((NC, RB, D), lambda i: (0, i, 0)),
            pl.BlockSpec((NC, RB, 1), lambda i: (0, i, 0)),
            pl.BlockSpec((1, D), lambda i: (0, 0)),
        ],
        out_specs=[
            pl.BlockSpec((RB, D), lambda i: (i, 0)),
            pl.BlockSpec((RB, D), lambda i: (i, 0)),
        ],
        out_shape=[
            jax.ShapeDtypeStruct((N, D), jnp.float32),
            jax.ShapeDtypeStruct((N, D), jnp.float32),
        ],
    )(agg, h2, deg, bias)


def kernel(view_a_pos, view_a_neg, view_b_pos, view_b_neg, x, W, b):
    sa = view_a_pos[0].astype(jnp.int32)
    da = view_a_pos[1].astype(jnp.int32)
    sb = view_b_pos[0].astype(jnp.int32)
    db = view_b_pos[1].astype(jnp.int32)
    src_t = jnp.stack([sa, sb + N]).reshape(NC, NS, NBLK, B)
    dst_t = jnp.stack([da, db]).reshape(NC, NS, NBLK, B)

    deg = _deg_sc(dst_t).reshape(NC, NPAD, 1)
    h2 = _h2_tc(x, W, deg)
    agg = _agg_sc(src_t, dst_t, h2.reshape(NC * N, D))
    xa, xb = _final_tc(agg, h2, deg, b.reshape(1, D))
    return (xa, xb)
